# baseline (speedup 1.0000x reference)
import jax
import jax.numpy as jnp
from jax import lax
from jax.experimental import pallas as pl
from jax.experimental.pallas import tpu as pltpu
from jax.experimental.pallas import tpu_sc as plsc

_N = 512
_H = 128
_P = 128
_G = 4
_GG = _G * _G
_NW = 32
_APW = _N // _NW
_L = 16
_NC = _N // _L
_CAP = _N + _L
_HC = _H // _L


def _pool_body(posx_hbm, posy_hbm, posx1_hbm, posy1_hbm, hid_hbm, out_hbm,
               posx_v, posy_v, posx1_v, posy1_v, hid_v, sl_v, sl2_v, acc_v):
    cid = lax.axis_index("c")
    sid = lax.axis_index("s")
    wid = sid * 2 + cid

    pltpu.sync_copy(posx_hbm, posx_v)
    pltpu.sync_copy(posy_hbm, posy_v)
    pltpu.sync_copy(posx1_hbm, posx1_v)
    pltpu.sync_copy(posy1_hbm, posy1_v)
    pltpu.sync_copy(hid_hbm, hid_v.at[pl.ds(0, _N)])

    for hc in range(_HC):
        hid_v[_N, pl.ds(hc * _L, _L)] = jnp.full((_L,), -3e38, jnp.float32)

    lanes = lax.iota(jnp.int32, _L)

    def per_agent(a, _):
        i = wid * _APW + a
        iv = jnp.full((_L,), i, jnp.int32)
        pxi = plsc.load_gather(posx1_v, [iv])
        pyi = plsc.load_gather(posy1_v, [iv])

        def chunk(jc, m):
            px = posx_v[jc, :]
            py = posy_v[jc, :]
            relx = px - pxi
            rely = py - pyi
            inb = (jnp.abs(relx) <= 1.0) & (jnp.abs(rely) <= 1.0)
            jv = jc * _L + lanes
            valid = inb & (jv != i)
            gx = jnp.clip((relx + 1.0) * 2.0, 0.0, _G - 1.0).astype(jnp.int32)
            gy = jnp.clip((rely + 1.0) * 2.0, 0.0, _G - 1.0).astype(jnp.int32)
            val = (gx * _G + gy) * _N + jv
            vi = valid.astype(jnp.int32)
            offs = m + plsc.cumsum(vi) - 1
            plsc.store_scatter(sl_v, [offs], val, mask=valid)
            return m + jnp.sum(vi)

        nv = lax.fori_loop(0, _NC, chunk, jnp.int32(0))

        plsc.store_scatter(sl_v, [nv + lanes],
                           jnp.full((_L,), _GG * _N, jnp.int32))

        def schunk(kc, offs):
            valv = sl_v[pl.ds(kc * _L, _L)]
            cv = valv >> 9
            jv = valv & (_N - 1)
            new = []
            for c in range(_GG):
                mc = cv == c
                cs = plsc.cumsum(mc.astype(jnp.int32))
                plsc.store_scatter(sl2_v, [offs[c] + cs - 1], jv, mask=mc)
                new.append(offs[c] + cs[_L - 1])
            return tuple(new)

        offs0 = tuple(jnp.int32(c * _CAP) for c in range(_GG))
        offs = lax.fori_loop(0, (nv + _L - 1) >> 4, schunk, offs0)

        for c in range(_GG):
            plsc.store_scatter(sl2_v, [offs[c] + lanes],
                               jnp.full((_L,), _N, jnp.int32))

        for c in range(_GG):
            cnt = offs[c] - c * _CAP

            def cchunk(kc, regs, c=c):
                jv16 = sl2_v[pl.ds(c * _CAP + kc * _L, _L)]
                nregs = list(regs)
                for l in range(_L):
                    j = jv16[l]
                    for hc in range(_HC):
                        nregs[hc] = jnp.maximum(
                            nregs[hc], hid_v[j, pl.ds(hc * _L, _L)])
                return tuple(nregs)

            init = tuple(jnp.full((_L,), -3e38, jnp.float32)
                         for _ in range(_HC))
            regs = lax.fori_loop(0, (cnt + _L - 1) >> 4, cchunk, init)
            nonempty = jnp.full((_L,), cnt, jnp.int32) > 0
            for hc in range(_HC):
                acc_v[c, pl.ds(hc * _L, _L)] = jnp.where(
                    nonempty, regs[hc], 0.0)

        pltpu.sync_copy(acc_v, out_hbm.at[i])
        return 0

    lax.fori_loop(0, _APW, per_agent, 0)


def _mm_body(g_ref, w_ref, b_ref, o_ref):
    o_ref[...] = jnp.maximum(
        jnp.dot(g_ref[...], w_ref[...], preferred_element_type=jnp.float32)
        + b_ref[...],
        0.0,
    )


def kernel(pos, hidden, W, b):
    posx = pos[:, 0].reshape(_NC, _L)
    posy = pos[:, 1].reshape(_NC, _L)

    grid = pl.kernel(
        _pool_body,
        out_type=jax.ShapeDtypeStruct((_N, _GG, _H), jnp.float32),
        mesh=plsc.VectorSubcoreMesh(core_axis_name="c", subcore_axis_name="s"),
        scratch_types=[
            pltpu.VMEM((_NC, _L), jnp.float32),
            pltpu.VMEM((_NC, _L), jnp.float32),
            pltpu.VMEM((_N,), jnp.float32),
            pltpu.VMEM((_N,), jnp.float32),
            pltpu.VMEM((_N + 1, _H), jnp.float32),
            pltpu.VMEM((_N + _L,), jnp.int32),
            pltpu.VMEM((_GG * _CAP,), jnp.int32),
            pltpu.VMEM((_GG, _H), jnp.float32),
        ],
        compiler_params=pltpu.CompilerParams(needs_layout_passes=False),
    )(posx, posy, posx.reshape(_N), posy.reshape(_N), hidden)

    return pl.pallas_call(
        _mm_body,
        out_shape=jax.ShapeDtypeStruct((_N, _P), jnp.float32),
    )(grid.reshape(_N, _GG * _H), W, b.reshape(1, _P))
